# baseline (speedup 1.0000x reference)
import functools

import jax
import jax.numpy as jnp
from jax import lax
from jax.experimental import pallas as pl
from jax.experimental.pallas import tpu as pltpu
from jax.experimental.pallas import tpu_sc as plsc

_BATCH = 16384
_HIST = 200
_EMBED = 64
_B = _BATCH * _HIST

_NC = 2
_NS = 16
_NW = _NC * _NS
_B_PER_W = _B // _NW
_CH = 1024
_NCHUNK = _B_PER_W // _CH

_mesh = plsc.VectorSubcoreMesh(core_axis_name="c", subcore_axis_name="s")


@functools.partial(
    pl.kernel,
    out_type=jax.ShapeDtypeStruct((_B, _EMBED), jnp.float32),
    mesh=_mesh,
    scratch_types=[
        pltpu.VMEM((_CH,), jnp.int32),
        pltpu.VMEM((_CH, _EMBED), jnp.float32),
        pltpu.SemaphoreType.DMA,
    ],
    compiler_params=pltpu.CompilerParams(use_tc_tiling_on_sc=False),
)
def _gather_kernel(codes_hbm, table_hbm, out_hbm, idx_v, rows_v, sem):
    wid = lax.axis_index("s") * _NC + lax.axis_index("c")
    base = wid * _B_PER_W

    @pl.loop(0, _NCHUNK)
    def _chunk(t):
        off = base + t * _CH
        pltpu.sync_copy(codes_hbm.at[pl.ds(off, _CH)], idx_v)
        pltpu.async_copy(table_hbm.at[idx_v], rows_v, sem).wait()
        pltpu.sync_copy(rows_v, out_hbm.at[pl.ds(off, _CH)])


def kernel(codes, table):
    flat = codes.reshape(-1).astype(jnp.int32)
    out = _gather_kernel(flat, table)
    return out.reshape(_BATCH, _HIST, _EMBED)


# speedup vs baseline: 1.0239x; 1.0239x over previous
import functools

import jax
import jax.numpy as jnp
from jax import lax
from jax.experimental import pallas as pl
from jax.experimental.pallas import tpu as pltpu
from jax.experimental.pallas import tpu_sc as plsc

_BATCH = 16384
_HIST = 200
_EMBED = 64
_B = _BATCH * _HIST

_NC = 2
_NS = 16
_NW = _NC * _NS
_B_PER_W = _B // _NW
_CH = 800
_NCHUNK = _B_PER_W // _CH

_mesh = plsc.VectorSubcoreMesh(core_axis_name="c", subcore_axis_name="s")


@functools.partial(
    pl.kernel,
    out_type=jax.ShapeDtypeStruct((_B, _EMBED), jnp.float32),
    mesh=_mesh,
    scratch_types=[
        pltpu.VMEM((_CH,), jnp.int32),
        pltpu.VMEM((_CH,), jnp.int32),
        pltpu.VMEM((_CH, _EMBED), jnp.float32),
        pltpu.VMEM((_CH, _EMBED), jnp.float32),
        pltpu.SemaphoreType.DMA,
        pltpu.SemaphoreType.DMA,
        pltpu.SemaphoreType.DMA,
        pltpu.SemaphoreType.DMA,
    ],
    compiler_params=pltpu.CompilerParams(use_tc_tiling_on_sc=False),
)
def _gather_kernel(codes_hbm, table_hbm, out_hbm, idx0, idx1, rows0, rows1,
                   gsem0, gsem1, ssem0, ssem1):
    wid = lax.axis_index("s") * _NC + lax.axis_index("c")
    base = wid * _B_PER_W

    def half(t, idx_v, rows_v, gsem, ssem, prev_outstanding):
        off = base + t * _CH
        pltpu.sync_copy(codes_hbm.at[pl.ds(off, _CH)], idx_v)

        @pl.when(prev_outstanding)
        def _():
            pltpu.make_async_copy(rows_v, out_hbm.at[pl.ds(off, _CH)], ssem).wait()

        pltpu.async_copy(table_hbm.at[idx_v], rows_v, gsem).wait()
        pltpu.async_copy(rows_v, out_hbm.at[pl.ds(off, _CH)], ssem)

    @pl.loop(0, _NCHUNK, step=2)
    def _chunks(t):
        half(t, idx0, rows0, gsem0, ssem0, t >= 2)
        half(t + 1, idx1, rows1, gsem1, ssem1, t >= 1)

    last = base + (_NCHUNK - 2) * _CH
    pltpu.make_async_copy(rows0, out_hbm.at[pl.ds(last, _CH)], ssem0).wait()
    pltpu.make_async_copy(rows1, out_hbm.at[pl.ds(last, _CH)], ssem1).wait()


def kernel(codes, table):
    flat = codes.reshape(-1).astype(jnp.int32)
    out = _gather_kernel(flat, table)
    return out.reshape(_BATCH, _HIST, _EMBED)
